# baseline (speedup 1.0000x reference)
import functools

import jax
import jax.numpy as jnp
from jax import lax
from jax.experimental import pallas as pl
from jax.experimental.pallas import tpu as pltpu
from jax.experimental.pallas import tpu_sc as plsc

N_VOCAB = 49408
N_EMBED = 768
N_TOKENS = 77
BATCH = 128

_NC = 2
_NS = 16
_NW = _NC * _NS
_ROWS_PER_W = BATCH // _NW
_LANES = 16
_PAD_TOKENS = 80
_CHUNK = 16
_NCHUNK = _PAD_TOKENS // _CHUNK
_PAIRS = N_EMBED // 32


def _make_sc_lookup():
    mesh = plsc.VectorSubcoreMesh(core_axis_name="c", subcore_axis_name="s")

    @functools.partial(
        pl.kernel,
        mesh=mesh,
        out_type=jax.ShapeDtypeStruct((BATCH, N_TOKENS, N_EMBED), jnp.float32),
        scratch_types=[
            pltpu.VMEM((_ROWS_PER_W, _PAD_TOKENS), jnp.int32),
            pltpu.VMEM((N_TOKENS * N_EMBED // 2,), jnp.int32),
            pltpu.VMEM((_CHUNK, N_EMBED), jnp.float32),
            pltpu.VMEM((_CHUNK, N_EMBED), jnp.float32),
            pltpu.VMEM((_CHUNK, N_EMBED), jnp.float32),
            pltpu.VMEM((N_TOKENS, N_EMBED), jnp.float32),
            pltpu.SemaphoreType.DMA,
            pltpu.SemaphoreType.DMA,
            pltpu.SemaphoreType.DMA,
        ],
    )
    def lookup(tok_hbm, table_hbm, pos_hbm, out_hbm,
               idx_v, pos_v, buf_a, buf_b, buf_c, wbuf,
               gsem, wsem, psem):
        wid = lax.axis_index("s") * _NC + lax.axis_index("c")
        pos_dma = pltpu.async_copy(pos_hbm, pos_v, psem)
        pltpu.sync_copy(tok_hbm.at[wid], idx_v)
        gbufs = (buf_a, buf_b, buf_c)

        def gather(b, p, buf):
            return pltpu.async_copy(
                table_hbm.at[idx_v.at[b, pl.ds(_CHUNK * p, _CHUNK)]],
                buf, gsem)

        def add_chunk(p, nrows):
            gbuf = gbufs[p % 3]

            @plsc.parallel_loop(0, nrows, unroll=8)
            def row_body(r):
                t = _CHUNK * p + r
                for j in range(_PAIRS):
                    off = pl.multiple_of(t * (N_EMBED // 2) + 16 * j, 16)
                    w = pos_v[pl.ds(off, _LANES)]
                    lo = lax.bitcast_convert_type(w << 16, jnp.float32)
                    hi = lax.bitcast_convert_type(w & jnp.int32(-65536),
                                                  jnp.float32)
                    sa = pl.ds(32 * j, _LANES)
                    sb = pl.ds(32 * j + _LANES, _LANES)
                    wbuf[t, sa] = gbuf[r, sa] + lo
                    wbuf[t, sb] = gbuf[r, sb] + hi

        pos_dma.wait()

        def wait_write():
            pltpu.make_async_copy(wbuf, out_hbm.at[0], wsem).wait()

        def batch_body(b, carry):
            def gat(p):
                return gather(b, p, gbufs[p % 3])

            handles = [gat(0), gat(1), gat(2)]

            @pl.when(b > 0)
            def _():
                wait_write()

            for p in range(_NCHUNK):
                handles[p].wait()
                add_chunk(p, _CHUNK if p + 1 < _NCHUNK
                          else N_TOKENS - _CHUNK * (_NCHUNK - 1))
                if p + 3 < _NCHUNK:
                    handles.append(gat(p + 3))
            pltpu.async_copy(wbuf, out_hbm.at[_ROWS_PER_W * wid + b], wsem)
            return carry

        lax.fori_loop(0, _ROWS_PER_W, batch_body, 0)
        wait_write()

    return lookup


_sc_lookup = _make_sc_lookup()


def kernel(tokens, token_embedding, position_embedding):
    tok32 = tokens.astype(jnp.int32).reshape(_NW, _ROWS_PER_W, N_TOKENS)
    tok32 = jnp.pad(tok32, ((0, 0), (0, 0), (0, _PAD_TOKENS - N_TOKENS)))
    pos_bf = (position_embedding.reshape(N_TOKENS, _PAIRS, 2, _LANES)
              .swapaxes(-2, -1)
              .reshape(N_TOKENS * N_EMBED // 2, 2)
              .astype(jnp.bfloat16))
    pos_pk = jax.lax.bitcast_convert_type(pos_bf, jnp.int32)
    return _sc_lookup(tok32, token_embedding, pos_pk)


# speedup vs baseline: 1.0100x; 1.0100x over previous
import functools

import jax
import jax.numpy as jnp
from jax import lax
from jax.experimental import pallas as pl
from jax.experimental.pallas import tpu as pltpu
from jax.experimental.pallas import tpu_sc as plsc

N_VOCAB = 49408
N_EMBED = 768
N_TOKENS = 77
BATCH = 128

_NC = 2
_NS = 16
_NW = _NC * _NS
_ROWS_PER_W = BATCH // _NW
_LANES = 16
_PAD_TOKENS = 80
_CHUNK = 16
_NCHUNK = _PAD_TOKENS // _CHUNK
_PAIRS = N_EMBED // 32


def _make_sc_lookup():
    mesh = plsc.VectorSubcoreMesh(core_axis_name="c", subcore_axis_name="s")

    @functools.partial(
        pl.kernel,
        mesh=mesh,
        out_type=jax.ShapeDtypeStruct((BATCH, N_TOKENS, N_EMBED), jnp.float32),
        scratch_types=[
            pltpu.VMEM((_ROWS_PER_W, _PAD_TOKENS), jnp.int32),
            pltpu.VMEM((N_TOKENS * N_EMBED // 2,), jnp.int32),
            pltpu.VMEM((_CHUNK, N_EMBED), jnp.float32),
            pltpu.VMEM((_CHUNK, N_EMBED), jnp.float32),
            pltpu.VMEM((_CHUNK, N_EMBED), jnp.float32),
            pltpu.VMEM((N_TOKENS, N_EMBED), jnp.float32),
            pltpu.SemaphoreType.DMA,
            pltpu.SemaphoreType.DMA,
            pltpu.SemaphoreType.DMA,
        ],
    )
    def lookup(tok_hbm, table_hbm, pos_hbm, out_hbm,
               idx_v, pos_v, buf_a, buf_b, buf_c, wbuf,
               gsem, wsem, psem):
        wid = lax.axis_index("s") * _NC + lax.axis_index("c")
        pos_dma = pltpu.async_copy(pos_hbm, pos_v, psem)
        pltpu.sync_copy(tok_hbm.at[wid], idx_v)
        gbufs = (buf_a, buf_b, buf_c)

        def gather(b, p, buf):
            return pltpu.async_copy(
                table_hbm.at[idx_v.at[b, pl.ds(_CHUNK * p, _CHUNK)]],
                buf, gsem)

        def add_chunk(p, nrows):
            gbuf = gbufs[p % 3]

            @plsc.parallel_loop(0, nrows, unroll=4)
            def row_body(r):
                t = _CHUNK * p + r
                for j in range(_PAIRS):
                    off = pl.multiple_of(t * (N_EMBED // 2) + 16 * j, 16)
                    w = pos_v[pl.ds(off, _LANES)]
                    lo = lax.bitcast_convert_type(w << 16, jnp.float32)
                    hi = lax.bitcast_convert_type(w & jnp.int32(-65536),
                                                  jnp.float32)
                    sa = pl.ds(32 * j, _LANES)
                    sb = pl.ds(32 * j + _LANES, _LANES)
                    wbuf[t, sa] = gbuf[r, sa] + lo
                    wbuf[t, sb] = gbuf[r, sb] + hi

        pos_dma.wait()

        def wait_write():
            pltpu.make_async_copy(wbuf, out_hbm.at[0], wsem).wait()

        def batch_body(b, carry):
            def gat(p):
                return gather(b, p, gbufs[p % 3])

            handles = [gat(0), gat(1), gat(2)]

            @pl.when(b > 0)
            def _():
                wait_write()

            for p in range(_NCHUNK):
                handles[p].wait()
                add_chunk(p, _CHUNK if p + 1 < _NCHUNK
                          else N_TOKENS - _CHUNK * (_NCHUNK - 1))
                if p + 3 < _NCHUNK:
                    handles.append(gat(p + 3))
            pltpu.async_copy(wbuf, out_hbm.at[_ROWS_PER_W * wid + b], wsem)
            return carry

        lax.fori_loop(0, _ROWS_PER_W, batch_body, 0)
        wait_write()

    return lookup


_sc_lookup = _make_sc_lookup()


def kernel(tokens, token_embedding, position_embedding):
    tok32 = tokens.astype(jnp.int32).reshape(_NW, _ROWS_PER_W, N_TOKENS)
    tok32 = jnp.pad(tok32, ((0, 0), (0, 0), (0, _PAD_TOKENS - N_TOKENS)))
    pos_bf = (position_embedding.reshape(N_TOKENS, _PAIRS, 2, _LANES)
              .swapaxes(-2, -1)
              .reshape(N_TOKENS * N_EMBED // 2, 2)
              .astype(jnp.bfloat16))
    pos_pk = jax.lax.bitcast_convert_type(pos_bf, jnp.int32)
    return _sc_lookup(tok32, token_embedding, pos_pk)


# speedup vs baseline: 1.0150x; 1.0049x over previous
import functools

import jax
import jax.numpy as jnp
from jax import lax
from jax.experimental import pallas as pl
from jax.experimental.pallas import tpu as pltpu
from jax.experimental.pallas import tpu_sc as plsc

N_VOCAB = 49408
N_EMBED = 768
N_TOKENS = 77
BATCH = 128

_NC = 2
_NS = 16
_NW = _NC * _NS
_ROWS_PER_W = BATCH // _NW
_LANES = 16
_PAD_TOKENS = 80
_CHUNK = 16
_NCHUNK = _PAD_TOKENS // _CHUNK
_PAIRS = N_EMBED // 32


def _make_sc_lookup():
    mesh = plsc.VectorSubcoreMesh(core_axis_name="c", subcore_axis_name="s")

    @functools.partial(
        pl.kernel,
        mesh=mesh,
        out_type=jax.ShapeDtypeStruct((BATCH, N_TOKENS, N_EMBED), jnp.float32),
        scratch_types=[
            pltpu.VMEM((_ROWS_PER_W, _PAD_TOKENS), jnp.int32),
            pltpu.VMEM((N_TOKENS * N_EMBED // 2,), jnp.int32),
            pltpu.VMEM((_CHUNK, N_EMBED), jnp.float32),
            pltpu.VMEM((_CHUNK, N_EMBED), jnp.float32),
            pltpu.VMEM((_CHUNK, N_EMBED), jnp.float32),
            pltpu.VMEM((N_TOKENS, N_EMBED), jnp.float32),
            pltpu.SemaphoreType.DMA,
            pltpu.SemaphoreType.DMA,
            pltpu.SemaphoreType.DMA,
            pltpu.SemaphoreType.DMA,
        ],
    )
    def lookup(tok_hbm, table_hbm, pos_hbm, out_hbm,
               idx_v, pos_v, buf_a, buf_b, buf_c, wbuf,
               gsem, wsem, wsem2, psem):
        wid = lax.axis_index("s") * _NC + lax.axis_index("c")
        pos_dma = pltpu.async_copy(pos_hbm, pos_v, psem)
        pltpu.sync_copy(tok_hbm.at[wid], idx_v)
        gbufs = (buf_a, buf_b, buf_c)

        def gather(b, p, buf):
            return pltpu.async_copy(
                table_hbm.at[idx_v.at[b, pl.ds(_CHUNK * p, _CHUNK)]],
                buf, gsem)

        def add_chunk(p, nrows):
            gbuf = gbufs[p % 3]

            @plsc.parallel_loop(0, nrows, unroll=4)
            def row_body(r):
                t = _CHUNK * p + r
                for j in range(_PAIRS):
                    off = pl.multiple_of(t * (N_EMBED // 2) + 16 * j, 16)
                    w = pos_v[pl.ds(off, _LANES)]
                    lo = lax.bitcast_convert_type(w << 16, jnp.float32)
                    hi = lax.bitcast_convert_type(w & jnp.int32(-65536),
                                                  jnp.float32)
                    sa = pl.ds(32 * j, _LANES)
                    sb = pl.ds(32 * j + _LANES, _LANES)
                    wbuf[t, sa] = gbuf[r, sa] + lo
                    wbuf[t, sb] = gbuf[r, sb] + hi

        pos_dma.wait()

        _H = N_EMBED // 2

        def wait_write():
            pltpu.make_async_copy(
                wbuf.at[:, pl.ds(0, _H)],
                out_hbm.at[0, :, pl.ds(0, _H)], wsem).wait()
            pltpu.make_async_copy(
                wbuf.at[:, pl.ds(_H, _H)],
                out_hbm.at[0, :, pl.ds(_H, _H)], wsem2).wait()

        def batch_body(b, carry):
            def gat(p):
                return gather(b, p, gbufs[p % 3])

            handles = [gat(0), gat(1), gat(2)]

            @pl.when(b > 0)
            def _():
                wait_write()

            for p in range(_NCHUNK):
                handles[p].wait()
                add_chunk(p, _CHUNK if p + 1 < _NCHUNK
                          else N_TOKENS - _CHUNK * (_NCHUNK - 1))
                if p + 3 < _NCHUNK:
                    handles.append(gat(p + 3))
            bb = _ROWS_PER_W * wid + b
            pltpu.async_copy(wbuf.at[:, pl.ds(0, _H)],
                             out_hbm.at[bb, :, pl.ds(0, _H)], wsem)
            pltpu.async_copy(wbuf.at[:, pl.ds(_H, _H)],
                             out_hbm.at[bb, :, pl.ds(_H, _H)], wsem2)
            return carry

        lax.fori_loop(0, _ROWS_PER_W, batch_body, 0)
        wait_write()

    return lookup


_sc_lookup = _make_sc_lookup()


def kernel(tokens, token_embedding, position_embedding):
    tok32 = tokens.astype(jnp.int32).reshape(_NW, _ROWS_PER_W, N_TOKENS)
    tok32 = jnp.pad(tok32, ((0, 0), (0, 0), (0, _PAD_TOKENS - N_TOKENS)))
    pos_bf = (position_embedding.reshape(N_TOKENS, _PAIRS, 2, _LANES)
              .swapaxes(-2, -1)
              .reshape(N_TOKENS * N_EMBED // 2, 2)
              .astype(jnp.bfloat16))
    pos_pk = jax.lax.bitcast_convert_type(pos_bf, jnp.int32)
    return _sc_lookup(tok32, token_embedding, pos_pk)
